# Initial kernel scaffold; baseline (speedup 1.0000x reference)
import functools
import jax
import jax.numpy as jnp
from jax import lax
from jax.experimental import pallas as pl
from jax.experimental.pallas import tpu as pltpu
from jax.experimental.pallas import tpu_sc as plsc


def _make_gather(N, V, D, num_cores, num_subcores):
    NW = num_cores * num_subcores
    b_per_w = N // NW
    C = 1600
    n_chunks = b_per_w // C

    mesh = plsc.VectorSubcoreMesh(core_axis_name="c", subcore_axis_name="s")

    @functools.partial(
        pl.kernel,
        mesh=mesh,
        out_type=jax.ShapeDtypeStruct((N, D), jnp.float32),
        scratch_types=[
            pltpu.VMEM((C,), jnp.int32),
            pltpu.VMEM((C, D), jnp.float32),
            pltpu.SemaphoreType.DMA,
        ],
    )
    def k(idx_hbm, table_hbm, out_hbm, idx_v, rows_v, sem):
        wid = lax.axis_index("s") * num_cores + lax.axis_index("c")
        base = wid * b_per_w

        def body(i, carry):
            off = pl.multiple_of(base + i * C, 8)
            pltpu.sync_copy(idx_hbm.at[pl.ds(off, C)], idx_v)
            pltpu.async_copy(table_hbm.at[idx_v], rows_v, sem).wait()
            pltpu.sync_copy(rows_v, out_hbm.at[pl.ds(off, C)])
            return carry

        lax.fori_loop(0, n_chunks, body, 0)

    return k


def kernel(x, table):
    B, S = x.shape
    V, D = table.shape
    N = B * S
    info = plsc.get_sparse_core_info()
    k = _make_gather(N, V, D, info.num_cores, info.num_subcores)
    out = k(x.reshape(N).astype(jnp.int32), table)
    return out.reshape(B, S, D)


# speedup vs baseline: 1.4761x; 1.4761x over previous
import functools
import jax
import jax.numpy as jnp
from jax import lax
from jax.experimental import pallas as pl
from jax.experimental.pallas import tpu as pltpu
from jax.experimental.pallas import tpu_sc as plsc


def _make_gather(N, V, D, num_cores, num_subcores):
    NW = num_cores * num_subcores
    b_per_w = N // NW
    C = 1600
    n_chunks = b_per_w // C

    mesh = plsc.VectorSubcoreMesh(core_axis_name="c", subcore_axis_name="s")

    @functools.partial(
        pl.kernel,
        mesh=mesh,
        out_type=jax.ShapeDtypeStruct((N, D), jnp.float32),
        scratch_types=[
            pltpu.VMEM((C,), jnp.int32),
            pltpu.VMEM((C, D), jnp.float32),
            pltpu.SemaphoreType.DMA,
        ],
        compiler_params=pltpu.CompilerParams(use_tc_tiling_on_sc=False),
    )
    def k(idx_hbm, table_hbm, out_hbm, idx_v, rows_v, sem):
        wid = lax.axis_index("s") * num_cores + lax.axis_index("c")
        base = wid * b_per_w

        def body(i, carry):
            off = pl.multiple_of(base + i * C, 8)
            pltpu.sync_copy(idx_hbm.at[pl.ds(off, C)], idx_v)
            pltpu.async_copy(table_hbm.at[idx_v], rows_v, sem).wait()
            pltpu.sync_copy(rows_v, out_hbm.at[pl.ds(off, C)])
            return carry

        lax.fori_loop(0, n_chunks, body, 0)

    return k


def kernel(x, table):
    B, S = x.shape
    V, D = table.shape
    N = B * S
    info = plsc.get_sparse_core_info()
    k = _make_gather(N, V, D, info.num_cores, info.num_subcores)
    out = k(x.reshape(N).astype(jnp.int32), table)
    return out.reshape(B, S, D)


# speedup vs baseline: 1.5024x; 1.0179x over previous
import functools
import jax
import jax.numpy as jnp
from jax import lax
from jax.experimental import pallas as pl
from jax.experimental.pallas import tpu as pltpu
from jax.experimental.pallas import tpu_sc as plsc


def _make_gather(N, V, D, num_cores, num_subcores):
    NW = num_cores * num_subcores
    b_per_w = N // NW
    C = 1280
    n_chunks = b_per_w // C

    mesh = plsc.VectorSubcoreMesh(core_axis_name="c", subcore_axis_name="s")

    @functools.partial(
        pl.kernel,
        mesh=mesh,
        out_type=jax.ShapeDtypeStruct((N, D), jnp.float32),
        scratch_types=[
            pltpu.VMEM((b_per_w,), jnp.int32),
            pltpu.VMEM((C, D), jnp.float32),
            pltpu.VMEM((C, D), jnp.float32),
            pltpu.SemaphoreType.DMA,
            pltpu.SemaphoreType.DMA,
        ],
        compiler_params=pltpu.CompilerParams(use_tc_tiling_on_sc=False),
    )
    def k(idx_hbm, table_hbm, out_hbm, idx_v, rows0, rows1, sem0, sem1):
        wid = lax.axis_index("s") * num_cores + lax.axis_index("c")
        base = wid * b_per_w
        pltpu.sync_copy(idx_hbm.at[pl.ds(base, b_per_w)], idx_v)

        rows = (rows0, rows1)
        sems = (sem0, sem1)

        def gather(c, b):
            return pltpu.async_copy(
                table_hbm.at[idx_v.at[pl.ds(c * C, C)]], rows[b], sems[b]
            )

        gather(0, 0)
        gather(1, 1)

        def body(i, carry):
            for b in range(2):
                c = 2 * i + b
                pltpu.make_async_copy(
                    table_hbm.at[idx_v.at[pl.ds(c * C, C)]], rows[b], sems[b]
                ).wait()
                off = pl.multiple_of(base + c * C, 8)
                pltpu.sync_copy(rows[b], out_hbm.at[pl.ds(off, C)])

                @pl.when(c + 2 < n_chunks)
                def _():
                    gather(c + 2, b)

            return carry

        lax.fori_loop(0, n_chunks // 2, body, 0)

    return k


def kernel(x, table):
    B, S = x.shape
    V, D = table.shape
    N = B * S
    info = plsc.get_sparse_core_info()
    k = _make_gather(N, V, D, info.num_cores, info.num_subcores)
    out = k(x.reshape(N).astype(jnp.int32), table)
    return out.reshape(B, S, D)
